# baseline (speedup 1.0000x reference)
import jax
import jax.numpy as jnp
from jax import lax
from jax.experimental import pallas as pl
from jax.experimental.pallas import tpu as pltpu

N = 1000000
NPAD = 7840 * 128
BLK = 80
GRID = 7840 // BLK
NEG_INF = float("-inf")
BIG = 10 ** 9


def _extract64(s_ref, bmv0, row_form):
    lane = lax.broadcasted_iota(jnp.int32, (1, 128), 1)
    ri = lax.broadcasted_iota(jnp.int32, (BLK, 128), 0)
    ci = lax.broadcasted_iota(jnp.int32, (BLK, 128), 1)
    fp = ri * 128 + ci
    if row_form:
        out0 = jnp.full((1, 64), NEG_INF, jnp.float32)
        oi = lax.broadcasted_iota(jnp.int32, (1, 64), 1)
    else:
        out0 = jnp.full((64, 1), NEG_INF, jnp.float32)
        oi = lax.broadcasted_iota(jnp.int32, (64, 1), 0)

    def body(k, carry):
        bmv, out = carry
        m = jnp.max(bmv)
        g = jnp.min(jnp.where(bmv == m, lane, BIG))
        blk = s_ref[g]
        p = jnp.min(jnp.where(blk == m, fp, BIG))
        blk2 = jnp.where(fp == p, NEG_INF, blk)
        s_ref[g] = blk2
        nm = jnp.max(blk2)
        bmv2 = jnp.where(lane == g, nm, bmv)
        out2 = jnp.where(oi == k, m, out)
        return bmv2, out2

    return lax.fori_loop(0, 64, body, (bmv0, out0))


def _body(s_ref, t_ref, out_ref, sneg_ref, spos_ref, bmn_ref, bmp_ref, acc_ref):
    i = pl.program_id(0)

    @pl.when(i == 0)
    def _():
        acc_ref[...] = jnp.zeros((BLK, 128), jnp.float32)
        bmn_ref[...] = jnp.full((1, 128), NEG_INF, jnp.float32)
        bmp_ref[...] = jnp.full((1, 128), NEG_INF, jnp.float32)

    sc = s_ref[0]
    t = t_ref[0]

    tf = t.astype(jnp.float32)
    sp = jnp.maximum(sc, 0.0) + jnp.log1p(jnp.exp(-jnp.abs(sc)))
    acc_ref[...] += jnp.where(t < 2, sp - tf * sc, 0.0)

    sneg = jnp.where(t == 0, sc, NEG_INF)
    spos = jnp.where(t == 1, -sc, NEG_INF)
    sneg_ref[i] = sneg
    spos_ref[i] = spos

    lane = lax.broadcasted_iota(jnp.int32, (1, 128), 1)
    bmn_ref[...] = jnp.where(lane == i, jnp.max(sneg), bmn_ref[...])
    bmp_ref[...] = jnp.where(lane == i, jnp.max(spos), bmp_ref[...])

    @pl.when(i == GRID - 1)
    def _finalize():
        _, hn_col = _extract64(sneg_ref, bmn_ref[...], row_form=False)
        _, hp_row = _extract64(spos_ref, bmp_ref[...], row_form=True)
        hard_pos_row = -hp_row
        diff = hn_col - hard_pos_row + jnp.float32(0.1)
        rank_sp = jnp.maximum(diff, 0.0) + jnp.log1p(jnp.exp(-jnp.abs(diff)))
        ranking = jnp.sum(rank_sp) / jnp.float32(64 * 64)
        ce = jnp.sum(acc_ref[...]) / jnp.float32(N)
        out_ref[0, 0] = jnp.float32(0.6) * ranking + jnp.float32(0.4) * ce


def kernel(inputs, targets):
    s = inputs[:, 1] - inputs[:, 0]
    sp = jnp.pad(s, (0, NPAD - N)).reshape(GRID, BLK, 128)
    tp = jnp.pad(targets.astype(jnp.int32), (0, NPAD - N),
                 constant_values=2).reshape(GRID, BLK, 128)
    out = pl.pallas_call(
        _body,
        grid=(GRID,),
        in_specs=[
            pl.BlockSpec((1, BLK, 128), lambda i: (i, 0, 0)),
            pl.BlockSpec((1, BLK, 128), lambda i: (i, 0, 0)),
        ],
        out_specs=pl.BlockSpec(memory_space=pltpu.SMEM,
                               block_shape=(1, 1), index_map=lambda i: (0, 0)),
        out_shape=jax.ShapeDtypeStruct((1, 1), jnp.float32),
        scratch_shapes=[
            pltpu.VMEM((GRID, BLK, 128), jnp.float32),
            pltpu.VMEM((GRID, BLK, 128), jnp.float32),
            pltpu.VMEM((1, 128), jnp.float32),
            pltpu.VMEM((1, 128), jnp.float32),
            pltpu.VMEM((BLK, 128), jnp.float32),
        ],
    )(sp, tp)
    return out[0, 0]


# speedup vs baseline: 1.2801x; 1.2801x over previous
import jax
import jax.numpy as jnp
from jax import lax
from jax.experimental import pallas as pl
from jax.experimental.pallas import tpu as pltpu

N = 1000000
NPAD = 7840 * 128
BLK = 80
GRID = 7840 // BLK
NEG_INF = float("-inf")
BIG = 10 ** 9


def _extract64(s_ref, bmv0, row_form):
    lane = lax.broadcasted_iota(jnp.int32, (1, 128), 1)
    ri = lax.broadcasted_iota(jnp.int32, (BLK, 128), 0)
    ci = lax.broadcasted_iota(jnp.int32, (BLK, 128), 1)
    fp = ri * 128 + ci
    if row_form:
        out0 = jnp.full((1, 64), NEG_INF, jnp.float32)
        oi = lax.broadcasted_iota(jnp.int32, (1, 64), 1)
    else:
        out0 = jnp.full((64, 1), NEG_INF, jnp.float32)
        oi = lax.broadcasted_iota(jnp.int32, (64, 1), 0)

    def body(k, carry):
        bmv, out = carry
        m = jnp.max(bmv)
        g = jnp.min(jnp.where(bmv == m, lane, BIG))
        blk = s_ref[g]
        p = jnp.min(jnp.where(blk == m, fp, BIG))
        blk2 = jnp.where(fp == p, NEG_INF, blk)
        s_ref[g] = blk2
        nm = jnp.max(blk2)
        bmv2 = jnp.where(lane == g, nm, bmv)
        out2 = jnp.where(oi == k, m, out)
        return bmv2, out2

    return lax.fori_loop(0, 64, body, (bmv0, out0))


def _body(s_ref, t_ref, out_ref, sneg_ref, spos_ref, bmn_ref, bmp_ref, acc_ref):
    i = pl.program_id(0)

    @pl.when(i == 0)
    def _():
        acc_ref[...] = jnp.zeros((BLK, 128), jnp.float32)
        bmn_ref[...] = jnp.full((1, 128), NEG_INF, jnp.float32)
        bmp_ref[...] = jnp.full((1, 128), NEG_INF, jnp.float32)

    sc = s_ref[0]
    t = t_ref[0]

    tf = t.astype(jnp.float32)
    sp = jnp.maximum(sc, 0.0) + jnp.log1p(jnp.exp(-jnp.abs(sc)))
    acc_ref[...] += jnp.where(t < 2, sp - tf * sc, 0.0)

    sneg = jnp.where(t == 0, sc, NEG_INF)
    spos = jnp.where(t == 1, -sc, NEG_INF)
    sneg_ref[i] = sneg
    spos_ref[i] = spos

    lane = lax.broadcasted_iota(jnp.int32, (1, 128), 1)
    bmn_ref[...] = jnp.where(lane == i, jnp.max(sneg), bmn_ref[...])
    bmp_ref[...] = jnp.where(lane == i, jnp.max(spos), bmp_ref[...])

    @pl.when(i == GRID - 1)
    def _finalize():
        lane = lax.broadcasted_iota(jnp.int32, (1, 128), 1)
        ri = lax.broadcasted_iota(jnp.int32, (BLK, 128), 0)
        ci = lax.broadcasted_iota(jnp.int32, (BLK, 128), 1)
        fp = ri * 128 + ci
        oc = lax.broadcasted_iota(jnp.int32, (64, 1), 0)
        orr = lax.broadcasted_iota(jnp.int32, (1, 64), 1)

        def pair_body(k, carry):
            bmn, bmp, hn, hp = carry
            mn = jnp.max(bmn)
            mp = jnp.max(bmp)
            gn = jnp.min(jnp.where(bmn == mn, lane, BIG))
            gp = jnp.min(jnp.where(bmp == mp, lane, BIG))
            bn = sneg_ref[gn]
            bp = spos_ref[gp]
            pn = jnp.min(jnp.where(bn == mn, fp, BIG))
            pp = jnp.min(jnp.where(bp == mp, fp, BIG))
            bn2 = jnp.where(fp == pn, NEG_INF, bn)
            bp2 = jnp.where(fp == pp, NEG_INF, bp)
            sneg_ref[gn] = bn2
            spos_ref[gp] = bp2
            bmn2 = jnp.where(lane == gn, jnp.max(bn2), bmn)
            bmp2 = jnp.where(lane == gp, jnp.max(bp2), bmp)
            hn2 = jnp.where(oc == k, mn, hn)
            hp2 = jnp.where(orr == k, mp, hp)
            return bmn2, bmp2, hn2, hp2

        _, _, hn_col, hp_row = lax.fori_loop(
            0, 64, pair_body,
            (bmn_ref[...], bmp_ref[...],
             jnp.full((64, 1), NEG_INF, jnp.float32),
             jnp.full((1, 64), NEG_INF, jnp.float32)))
        hard_pos_row = -hp_row
        diff = hn_col - hard_pos_row + jnp.float32(0.1)
        rank_sp = jnp.maximum(diff, 0.0) + jnp.log1p(jnp.exp(-jnp.abs(diff)))
        ranking = jnp.sum(rank_sp) / jnp.float32(64 * 64)
        ce = jnp.sum(acc_ref[...]) / jnp.float32(N)
        out_ref[0, 0] = jnp.float32(0.6) * ranking + jnp.float32(0.4) * ce


def kernel(inputs, targets):
    s = inputs[:, 1] - inputs[:, 0]
    sp = jnp.pad(s, (0, NPAD - N)).reshape(GRID, BLK, 128)
    tp = jnp.pad(targets.astype(jnp.int32), (0, NPAD - N),
                 constant_values=2).reshape(GRID, BLK, 128)
    out = pl.pallas_call(
        _body,
        grid=(GRID,),
        in_specs=[
            pl.BlockSpec((1, BLK, 128), lambda i: (i, 0, 0)),
            pl.BlockSpec((1, BLK, 128), lambda i: (i, 0, 0)),
        ],
        out_specs=pl.BlockSpec(memory_space=pltpu.SMEM,
                               block_shape=(1, 1), index_map=lambda i: (0, 0)),
        out_shape=jax.ShapeDtypeStruct((1, 1), jnp.float32),
        scratch_shapes=[
            pltpu.VMEM((GRID, BLK, 128), jnp.float32),
            pltpu.VMEM((GRID, BLK, 128), jnp.float32),
            pltpu.VMEM((1, 128), jnp.float32),
            pltpu.VMEM((1, 128), jnp.float32),
            pltpu.VMEM((BLK, 128), jnp.float32),
        ],
    )(sp, tp)
    return out[0, 0]
